# baseline (speedup 1.0000x reference)
import functools

import jax
import jax.numpy as jnp
from jax import lax
from jax.experimental import pallas as pl
from jax.experimental.pallas import tpu as pltpu
from jax.experimental.pallas import tpu_sc as plsc

_NC = 2
_NS = 16
_NW = _NC * _NS

_K = 25
_GA = 4
_RB = _GA
_IB = 2 * _RB
_FA = _IB


def _segment_sum_sc(x, src, dst, zeros):
    n, d = x.shape
    e = src.shape[0]
    epw = e // _NW
    nchunk = epw // _K
    rows_per_tile = (n // _NS) // 8 * 8
    rows_tail = n - rows_per_tile * _NS

    src3 = src.reshape(_NW, nchunk, _K)
    dst3 = dst.reshape(_NW, nchunk, _K)

    mesh = plsc.VectorSubcoreMesh(core_axis_name="c", subcore_axis_name="s")

    @functools.partial(
        pl.kernel,
        out_type=jax.ShapeDtypeStruct((2 * n, d), jnp.float32),
        mesh=mesh,
        scratch_types=[
            pltpu.VMEM_SHARED((n, d), jnp.float32),
            pltpu.VMEM((_IB, _K), jnp.int32),
            pltpu.VMEM((_IB, _K), jnp.int32),
            pltpu.VMEM((_RB, _K, d), jnp.float32),
            [pltpu.SemaphoreType.DMA] * _IB,
            [pltpu.SemaphoreType.DMA] * _RB,
            [pltpu.SemaphoreType.DMA] * _RB,
        ],
    )
    def segsum(x_hbm, src_hbm, dst_hbm, zero_hbm, out_hbm,
               acc_sh, sidx_v, didx_v, ring_v, isem, gsem, ssem):
        cid = lax.axis_index("c")
        sid = lax.axis_index("s")
        wid = sid * _NC + cid

        r0 = sid * rows_per_tile
        pltpu.sync_copy(zero_hbm.at[pl.ds(r0, rows_per_tile)],
                        acc_sh.at[pl.ds(r0, rows_per_tile)])
        if rows_tail:
            @pl.when(sid == _NS - 1)
            def _zero_tail():
                t0 = rows_per_tile * _NS
                pltpu.sync_copy(zero_hbm.at[pl.ds(t0, rows_tail)],
                                acc_sh.at[pl.ds(t0, rows_tail)])
        plsc.subcore_barrier()

        def fetch_idx(c, j):
            pltpu.make_async_copy(src_hbm.at[wid, c], sidx_v.at[j],
                                  isem[j]).start()
            pltpu.make_async_copy(dst_hbm.at[wid, c], didx_v.at[j],
                                  isem[j]).start()

        def wait_idx(c, j):
            pltpu.make_async_copy(src_hbm.at[wid, c], sidx_v.at[j],
                                  isem[j]).wait()
            pltpu.make_async_copy(dst_hbm.at[wid, c], didx_v.at[j],
                                  isem[j]).wait()

        def gather(b, j):
            return pltpu.make_async_copy(
                x_hbm.at[sidx_v.at[j]], ring_v.at[b], gsem[b])

        def scatter_start(b, j):
            pltpu.async_copy(
                ring_v.at[b], acc_sh.at[didx_v.at[j]], ssem[b], add=True)

        def scatter_drain(b, j):
            pltpu.make_async_copy(
                ring_v.at[b], acc_sh.at[didx_v.at[j]], ssem[b]).wait()

        for j in range(_FA):
            fetch_idx(j, j)
        for c in range(_GA):
            wait_idx(c, c)
            gather(c, c).start()

        @pl.loop(0, nchunk, step=_IB)
        def _chunks(g):
            for u in range(_IB):
                c = g + u
                b = u % _RB
                gather(b, u).wait()
                scatter_start(b, u)
                scatter_drain(b, u)
                rc = c + _GA
                rb = (u + _GA) % _RB
                rj = (u + _GA) % _IB

                @pl.when(rc < nchunk)
                def _refill():
                    wait_idx(rc, rj)
                    gather(rb, rj).start()

                fq = c + _FA
                fj = (u + _FA) % _IB

                @pl.when(fq < nchunk)
                def _next_fetch():
                    fetch_idx(fq, fj)

        plsc.subcore_barrier()

        pltpu.sync_copy(acc_sh.at[pl.ds(r0, rows_per_tile)],
                        out_hbm.at[pl.ds(cid * n + r0, rows_per_tile)])
        if rows_tail:
            @pl.when(sid == _NS - 1)
            def _write_tail():
                t0 = rows_per_tile * _NS
                pltpu.sync_copy(acc_sh.at[pl.ds(t0, rows_tail)],
                                out_hbm.at[pl.ds(cid * n + t0, rows_tail)])

    return segsum(x, src3, dst3, zeros)


def _mlp_bn_body(x_ref, p_ref, wa_ref, ba_ref, wb_ref, bb_ref, g_ref, be_ref,
                 o_ref):
    n = x_ref.shape[0]
    h0 = x_ref[...] + p_ref[:n, :] + p_ref[n:, :]
    t = jnp.dot(h0, wa_ref[...], preferred_element_type=jnp.float32)
    t = jnp.maximum(t + ba_ref[...], 0.0)
    u = jnp.dot(t, wb_ref[...], preferred_element_type=jnp.float32)
    u = jnp.maximum(u + bb_ref[...], 0.0)
    mu = jnp.mean(u, axis=0, keepdims=True)
    dev = u - mu
    var = jnp.mean(dev * dev, axis=0, keepdims=True)
    o_ref[...] = dev * lax.rsqrt(var + 1e-5) * g_ref[...] + be_ref[...]


def _mlp_bn(x, parts, wa, ba, wb, bb, g, be):
    n, d = x.shape
    return pl.pallas_call(
        _mlp_bn_body,
        out_shape=jax.ShapeDtypeStruct((n, d), jnp.float32),
    )(x, parts, wa, ba.reshape(1, d), wb, bb.reshape(1, -1),
      g.reshape(1, -1), be.reshape(1, -1))


def kernel(x, edge_index, W1a, b1a, W1b, b1b, g1, be1,
           W5a, b5a, W5b, b5b, g5, be5):
    n, d = x.shape
    out_dim = W5b.shape[1]
    src = edge_index[0]
    dst = edge_index[1]
    zeros = jnp.zeros_like(x)

    parts1 = _segment_sum_sc(x, src, dst, zeros)
    h = _mlp_bn(x, parts1, W1a, b1a, W1b, b1b, g1, be1)

    pad = d - out_dim
    w5b_p = jnp.pad(W5b, ((0, 0), (0, pad)))
    b5b_p = jnp.pad(b5b, (0, pad))
    g5_p = jnp.pad(g5, (0, pad))
    be5_p = jnp.pad(be5, (0, pad))

    parts2 = _segment_sum_sc(h, src, dst, zeros)
    h2 = _mlp_bn(h, parts2, W5a, b5a, w5b_p, b5b_p, g5_p, be5_p)
    return h2[:, :out_dim]


# speedup vs baseline: 1.1694x; 1.1694x over previous
import functools

import jax
import jax.numpy as jnp
from jax import lax
from jax.experimental import pallas as pl
from jax.experimental.pallas import tpu as pltpu
from jax.experimental.pallas import tpu_sc as plsc

_NC = 2
_NS = 16
_NW = _NC * _NS

_K = 125
_GA = 2
_RB = _GA
_IB = 2 * _RB
_FA = _IB


def _segment_sum_sc(x, src, dst, zeros):
    n, d = x.shape
    e = src.shape[0]
    epw = e // _NW
    nchunk = epw // _K
    rows_per_tile = (n // _NS) // 8 * 8
    rows_tail = n - rows_per_tile * _NS

    src3 = src.reshape(_NW, nchunk, _K)
    dst3 = dst.reshape(_NW, nchunk, _K)

    mesh = plsc.VectorSubcoreMesh(core_axis_name="c", subcore_axis_name="s")

    @functools.partial(
        pl.kernel,
        out_type=jax.ShapeDtypeStruct((2 * n, d), jnp.float32),
        mesh=mesh,
        scratch_types=[
            pltpu.VMEM_SHARED((n, d), jnp.float32),
            pltpu.VMEM((_IB, _K), jnp.int32),
            pltpu.VMEM((_IB, _K), jnp.int32),
            pltpu.VMEM((_RB, _K, d), jnp.float32),
            [pltpu.SemaphoreType.DMA] * _IB,
            [pltpu.SemaphoreType.DMA] * _RB,
            [pltpu.SemaphoreType.DMA] * _RB,
        ],
    )
    def segsum(x_hbm, src_hbm, dst_hbm, zero_hbm, out_hbm,
               acc_sh, sidx_v, didx_v, ring_v, isem, gsem, ssem):
        cid = lax.axis_index("c")
        sid = lax.axis_index("s")
        wid = sid * _NC + cid

        r0 = sid * rows_per_tile
        pltpu.sync_copy(zero_hbm.at[pl.ds(r0, rows_per_tile)],
                        acc_sh.at[pl.ds(r0, rows_per_tile)])
        if rows_tail:
            @pl.when(sid == _NS - 1)
            def _zero_tail():
                t0 = rows_per_tile * _NS
                pltpu.sync_copy(zero_hbm.at[pl.ds(t0, rows_tail)],
                                acc_sh.at[pl.ds(t0, rows_tail)])
        plsc.subcore_barrier()

        def fetch_idx(c, j):
            pltpu.make_async_copy(src_hbm.at[wid, c], sidx_v.at[j],
                                  isem[j]).start()
            pltpu.make_async_copy(dst_hbm.at[wid, c], didx_v.at[j],
                                  isem[j]).start()

        def wait_idx(c, j):
            pltpu.make_async_copy(src_hbm.at[wid, c], sidx_v.at[j],
                                  isem[j]).wait()
            pltpu.make_async_copy(dst_hbm.at[wid, c], didx_v.at[j],
                                  isem[j]).wait()

        def gather(b, j):
            return pltpu.make_async_copy(
                x_hbm.at[sidx_v.at[j]], ring_v.at[b], gsem[b])

        def scatter_start(b, j):
            pltpu.async_copy(
                ring_v.at[b], acc_sh.at[didx_v.at[j]], ssem[b], add=True)

        def scatter_drain(b, j):
            pltpu.make_async_copy(
                ring_v.at[b], acc_sh.at[didx_v.at[j]], ssem[b]).wait()

        for j in range(_FA):
            fetch_idx(j, j)
        for c in range(_GA):
            wait_idx(c, c)
            gather(c, c).start()

        @pl.loop(0, nchunk, step=_IB)
        def _chunks(g):
            for u in range(_IB):
                c = g + u
                b = u % _RB
                gather(b, u).wait()
                scatter_start(b, u)
                scatter_drain(b, u)
                rc = c + _GA
                rb = (u + _GA) % _RB
                rj = (u + _GA) % _IB

                @pl.when(rc < nchunk)
                def _refill():
                    wait_idx(rc, rj)
                    gather(rb, rj).start()

                fq = c + _FA
                fj = (u + _FA) % _IB

                @pl.when(fq < nchunk)
                def _next_fetch():
                    fetch_idx(fq, fj)

        plsc.subcore_barrier()

        pltpu.sync_copy(acc_sh.at[pl.ds(r0, rows_per_tile)],
                        out_hbm.at[pl.ds(cid * n + r0, rows_per_tile)])
        if rows_tail:
            @pl.when(sid == _NS - 1)
            def _write_tail():
                t0 = rows_per_tile * _NS
                pltpu.sync_copy(acc_sh.at[pl.ds(t0, rows_tail)],
                                out_hbm.at[pl.ds(cid * n + t0, rows_tail)])

    return segsum(x, src3, dst3, zeros)


def _mlp_bn_body(x_ref, p_ref, wa_ref, ba_ref, wb_ref, bb_ref, g_ref, be_ref,
                 o_ref):
    n = x_ref.shape[0]
    h0 = x_ref[...] + p_ref[:n, :] + p_ref[n:, :]
    t = jnp.dot(h0, wa_ref[...], preferred_element_type=jnp.float32)
    t = jnp.maximum(t + ba_ref[...], 0.0)
    u = jnp.dot(t, wb_ref[...], preferred_element_type=jnp.float32)
    u = jnp.maximum(u + bb_ref[...], 0.0)
    mu = jnp.mean(u, axis=0, keepdims=True)
    dev = u - mu
    var = jnp.mean(dev * dev, axis=0, keepdims=True)
    o_ref[...] = dev * lax.rsqrt(var + 1e-5) * g_ref[...] + be_ref[...]


def _mlp_bn(x, parts, wa, ba, wb, bb, g, be):
    n, d = x.shape
    return pl.pallas_call(
        _mlp_bn_body,
        out_shape=jax.ShapeDtypeStruct((n, d), jnp.float32),
    )(x, parts, wa, ba.reshape(1, d), wb, bb.reshape(1, -1),
      g.reshape(1, -1), be.reshape(1, -1))


def kernel(x, edge_index, W1a, b1a, W1b, b1b, g1, be1,
           W5a, b5a, W5b, b5b, g5, be5):
    n, d = x.shape
    out_dim = W5b.shape[1]
    src = edge_index[0]
    dst = edge_index[1]
    zeros = jnp.zeros_like(x)

    parts1 = _segment_sum_sc(x, src, dst, zeros)
    h = _mlp_bn(x, parts1, W1a, b1a, W1b, b1b, g1, be1)

    pad = d - out_dim
    w5b_p = jnp.pad(W5b, ((0, 0), (0, pad)))
    b5b_p = jnp.pad(b5b, (0, pad))
    g5_p = jnp.pad(g5, (0, pad))
    be5_p = jnp.pad(be5, (0, pad))

    parts2 = _segment_sum_sc(h, src, dst, zeros)
    h2 = _mlp_bn(h, parts2, W5a, b5a, w5b_p, b5b_p, g5_p, be5_p)
    return h2[:, :out_dim]


# speedup vs baseline: 1.2864x; 1.1001x over previous
import functools

import jax
import jax.numpy as jnp
from jax import lax
from jax.experimental import pallas as pl
from jax.experimental.pallas import tpu as pltpu
from jax.experimental.pallas import tpu_sc as plsc

_NC = 2
_NS = 16
_NW = _NC * _NS

_K = 50
_GA = 4
_RB = _GA
_IB = 2 * _RB
_FA = _IB


def _segment_sum_sc(x, src, dst):
    n, d = x.shape
    e = src.shape[0]
    epw = e // _NW
    nchunk = epw // _K
    rows_per_tile = (n // _NS) // 8 * 8
    rows_tail = n - rows_per_tile * _NS

    src3 = src.reshape(_NW, nchunk, _K)
    dst3 = dst.reshape(_NW, nchunk, _K)

    mesh = plsc.VectorSubcoreMesh(core_axis_name="c", subcore_axis_name="s")

    zrows = 64

    @functools.partial(
        pl.kernel,
        out_type=jax.ShapeDtypeStruct((2 * n, d), jnp.float32),
        mesh=mesh,
        scratch_types=[
            pltpu.VMEM_SHARED((n, d), jnp.float32),
            pltpu.VMEM((_IB, _K), jnp.int32),
            pltpu.VMEM((_IB, _K), jnp.int32),
            pltpu.VMEM((_RB, _K, d), jnp.float32),
            pltpu.VMEM((zrows, d), jnp.float32),
            [pltpu.SemaphoreType.DMA] * _IB,
            [pltpu.SemaphoreType.DMA] * _RB,
            [pltpu.SemaphoreType.DMA] * _RB,
        ],
    )
    def segsum(x_hbm, src_hbm, dst_hbm, out_hbm,
               acc_sh, sidx_v, didx_v, ring_v, zbuf_v, isem, gsem, ssem):
        cid = lax.axis_index("c")
        sid = lax.axis_index("s")
        wid = sid * _NC + cid

        @pl.loop(0, zrows)
        def _zrow(r):
            for cc in range(d // 16):
                zbuf_v[r, pl.ds(cc * 16, 16)] = jnp.zeros((16,), jnp.float32)

        r0 = sid * rows_per_tile
        full, part = divmod(rows_per_tile, zrows)
        for z in range(full):
            pltpu.sync_copy(zbuf_v,
                            acc_sh.at[pl.ds(r0 + z * zrows, zrows)])
        if part:
            pltpu.sync_copy(zbuf_v.at[pl.ds(0, part)],
                            acc_sh.at[pl.ds(r0 + full * zrows, part)])
        if rows_tail:
            @pl.when(sid == _NS - 1)
            def _zero_tail():
                t0 = rows_per_tile * _NS
                pltpu.sync_copy(zbuf_v.at[pl.ds(0, rows_tail)],
                                acc_sh.at[pl.ds(t0, rows_tail)])
        plsc.subcore_barrier()

        def fetch_idx(c, j):
            pltpu.make_async_copy(src_hbm.at[wid, c], sidx_v.at[j],
                                  isem[j]).start()
            pltpu.make_async_copy(dst_hbm.at[wid, c], didx_v.at[j],
                                  isem[j]).start()

        def wait_idx(c, j):
            pltpu.make_async_copy(src_hbm.at[wid, c], sidx_v.at[j],
                                  isem[j]).wait()
            pltpu.make_async_copy(dst_hbm.at[wid, c], didx_v.at[j],
                                  isem[j]).wait()

        def gather(b, j):
            return pltpu.make_async_copy(
                x_hbm.at[sidx_v.at[j]], ring_v.at[b], gsem[b])

        def scatter_start(b, j):
            pltpu.async_copy(
                ring_v.at[b], acc_sh.at[didx_v.at[j]], ssem[b], add=True)

        def scatter_drain(b, j):
            pltpu.make_async_copy(
                ring_v.at[b], acc_sh.at[didx_v.at[j]], ssem[b]).wait()

        for j in range(_FA):
            fetch_idx(j, j)
        for c in range(_GA):
            wait_idx(c, c)
            gather(c, c).start()

        @pl.loop(0, nchunk, step=_IB)
        def _chunks(g):
            for u in range(_IB):
                c = g + u
                b = u % _RB
                gather(b, u).wait()
                scatter_start(b, u)
                scatter_drain(b, u)
                rc = c + _GA
                rb = (u + _GA) % _RB
                rj = (u + _GA) % _IB

                @pl.when(rc < nchunk)
                def _refill():
                    wait_idx(rc, rj)
                    gather(rb, rj).start()

                fq = c + _FA
                fj = (u + _FA) % _IB

                @pl.when(fq < nchunk)
                def _next_fetch():
                    fetch_idx(fq, fj)

        plsc.subcore_barrier()

        pltpu.sync_copy(acc_sh.at[pl.ds(r0, rows_per_tile)],
                        out_hbm.at[pl.ds(cid * n + r0, rows_per_tile)])
        if rows_tail:
            @pl.when(sid == _NS - 1)
            def _write_tail():
                t0 = rows_per_tile * _NS
                pltpu.sync_copy(acc_sh.at[pl.ds(t0, rows_tail)],
                                out_hbm.at[pl.ds(cid * n + t0, rows_tail)])

    return segsum(x, src3, dst3)


def _mlp_bn_body(x_ref, p_ref, wa_ref, ba_ref, wb_ref, bb_ref, g_ref, be_ref,
                 o_ref):
    n = x_ref.shape[0]
    od = o_ref.shape[1]
    h0 = x_ref[...] + p_ref[:n, :] + p_ref[n:, :]
    t = jnp.dot(h0, wa_ref[...], preferred_element_type=jnp.float32)
    t = jnp.maximum(t + ba_ref[...], 0.0)
    u = jnp.dot(t, wb_ref[...], preferred_element_type=jnp.float32)
    u = jnp.maximum(u + bb_ref[...], 0.0)
    mu = jnp.mean(u, axis=0, keepdims=True)
    dev = u - mu
    var = jnp.mean(dev * dev, axis=0, keepdims=True)
    res = dev * lax.rsqrt(var + 1e-5) * g_ref[...] + be_ref[...]
    o_ref[...] = res[:, :od]


def _mlp_bn(x, parts, wa, ba, wb, bb, g, be, out_dim):
    n, d = x.shape
    return pl.pallas_call(
        _mlp_bn_body,
        out_shape=jax.ShapeDtypeStruct((n, out_dim), jnp.float32),
    )(x, parts, wa, ba.reshape(1, d), wb, bb.reshape(1, -1),
      g.reshape(1, -1), be.reshape(1, -1))


def kernel(x, edge_index, W1a, b1a, W1b, b1b, g1, be1,
           W5a, b5a, W5b, b5b, g5, be5):
    n, d = x.shape
    out_dim = W5b.shape[1]
    src = edge_index[0]
    dst = edge_index[1]

    parts1 = _segment_sum_sc(x, src, dst)
    h = _mlp_bn(x, parts1, W1a, b1a, W1b, b1b, g1, be1, d)

    pad = d - out_dim
    w5b_p = jnp.pad(W5b, ((0, 0), (0, pad)))
    b5b_p = jnp.pad(b5b, (0, pad))
    g5_p = jnp.pad(g5, (0, pad))
    be5_p = jnp.pad(be5, (0, pad))

    parts2 = _segment_sum_sc(h, src, dst)
    return _mlp_bn(h, parts2, W5a, b5a, w5b_p, b5b_p, g5_p, be5_p, out_dim)
